# Initial kernel scaffold; baseline (speedup 1.0000x reference)
import jax
import jax.numpy as jnp
from jax.experimental import pallas as pl


def kernel(hidden_states, top_k_weights, gate_proj, up_proj, down_proj, top_k_index):
    raise NotImplementedError("write your pallas kernel here")


# speedup vs baseline: 2.0458x; 2.0458x over previous
import functools
import jax
import jax.numpy as jnp
from jax.experimental import pallas as pl
from jax.experimental.pallas import tpu as pltpu


def _ffn_dense_kernel(tw_ref, ti_ref, x_ref, wg_ref, wu_ref, wd_ref, out_ref):
    e = pl.program_id(0)
    fb = pl.program_id(1)

    @pl.when((e == 0) & (fb == 0))
    def _init():
        out_ref[...] = jnp.zeros_like(out_ref)

    x = x_ref[...]
    gate = jax.lax.dot_general(x, wg_ref[0], (((1,), (1,)), ((), ())),
                               preferred_element_type=jnp.float32)
    up = jax.lax.dot_general(x, wu_ref[0], (((1,), (1,)), ((), ())),
                             preferred_element_type=jnp.float32)
    h = gate * jax.lax.logistic(gate) * up
    y = jax.lax.dot_general(h, wd_ref[0], (((1,), (1,)), ((), ())),
                            preferred_element_type=jnp.float32)
    w = jnp.sum(jnp.where(ti_ref[...] == e, tw_ref[...], 0.0), axis=1)
    out_ref[...] += y * w[:, None]


def kernel(hidden_states, top_k_weights, gate_proj, up_proj, down_proj, top_k_index):
    T, Dm = hidden_states.shape
    E, F, _ = gate_proj.shape
    K = top_k_index.shape[1]
    BF = 256
    ti = top_k_index.astype(jnp.int32)

    out = pl.pallas_call(
        _ffn_dense_kernel,
        grid=(E, F // BF),
        in_specs=[
            pl.BlockSpec((T, K), lambda e, f: (0, 0)),
            pl.BlockSpec((T, K), lambda e, f: (0, 0)),
            pl.BlockSpec((T, Dm), lambda e, f: (0, 0)),
            pl.BlockSpec((1, BF, Dm), lambda e, f: (e, f, 0)),
            pl.BlockSpec((1, BF, Dm), lambda e, f: (e, f, 0)),
            pl.BlockSpec((1, Dm, BF), lambda e, f: (e, 0, f)),
        ],
        out_specs=pl.BlockSpec((T, Dm), lambda e, f: (0, 0)),
        out_shape=jax.ShapeDtypeStruct((T, Dm), jnp.float32),
        compiler_params=pltpu.CompilerParams(
            dimension_semantics=("arbitrary", "arbitrary"),
        ),
    )(top_k_weights, ti, hidden_states, gate_proj, up_proj, down_proj)
    return out
